# baseline (speedup 1.0000x reference)
import functools

import jax
import jax.numpy as jnp
from jax import lax
from jax.experimental import pallas as pl
from jax.experimental.pallas import tpu as pltpu
from jax.experimental.pallas import tpu_sc as plsc

_B = 16384
_D = 64
_V = 1000000
_NC = 2
_NS = 16
_NW = _NC * _NS
_BPW = _B // _NW

_TILE = 1024
_EPS = 1e-5


def _sc_gather_pair(ut2, it2, u2, i2):
    mesh = plsc.VectorSubcoreMesh(core_axis_name="c", subcore_axis_name="s")
    out_t = (jax.ShapeDtypeStruct((_B, 2 * _D), jnp.float32),
             jax.ShapeDtypeStruct((_B, 2 * _D), jnp.float32))

    half = _BPW // 2

    @functools.partial(
        pl.kernel, mesh=mesh, out_type=out_t,
        scratch_types=[
            pltpu.VMEM((_BPW,), jnp.int32),
            pltpu.VMEM((_BPW,), jnp.int32),
            pltpu.VMEM((half, 2 * _D), jnp.float32),
            pltpu.VMEM((half, 2 * _D), jnp.float32),
            pltpu.SemaphoreType.DMA,
            pltpu.SemaphoreType.DMA,
        ],
    )
    def k(ut_hbm, it_hbm, u_hbm, i_hbm, ou_hbm, oi_hbm,
          uidx_v, iidx_v, rows0_v, rows1_v, sem0, sem1):
        wid = lax.axis_index("s") * _NC + lax.axis_index("c")
        base = wid * _BPW
        pltpu.sync_copy(u_hbm.at[pl.ds(base, _BPW)], uidx_v)
        pltpu.sync_copy(i_hbm.at[pl.ds(base, _BPW)], iidx_v)
        c0 = pltpu.make_async_copy(
            ut_hbm.at[uidx_v.at[pl.ds(0, half)]], rows0_v, sem0)
        c1 = pltpu.make_async_copy(
            ut_hbm.at[uidx_v.at[pl.ds(half, half)]], rows1_v, sem1)
        c0.start()
        c1.start()
        c0.wait()
        pltpu.sync_copy(rows0_v, ou_hbm.at[pl.ds(base, half)])
        c2 = pltpu.make_async_copy(
            it_hbm.at[iidx_v.at[pl.ds(0, half)]], rows0_v, sem0)
        c2.start()
        c1.wait()
        pltpu.sync_copy(rows1_v, ou_hbm.at[pl.ds(base + half, half)])
        c3 = pltpu.make_async_copy(
            it_hbm.at[iidx_v.at[pl.ds(half, half)]], rows1_v, sem1)
        c3.start()
        c2.wait()
        pltpu.sync_copy(rows0_v, oi_hbm.at[pl.ds(base, half)])
        c3.wait()
        pltpu.sync_copy(rows1_v, oi_hbm.at[pl.ds(base + half, half)])

    return k(ut2, it2, u2, i2)


def _ln_relu(x, g, b):
    mu = jnp.mean(x, axis=-1, keepdims=True)
    xc = x - mu
    var = jnp.mean(xc * xc, axis=-1, keepdims=True)
    y = xc * lax.rsqrt(var + _EPS) * g + b
    return jnp.maximum(y, 0.0)


def _mlp_body(pu_ref, pi_ref, mu_ref, mi_ref,
              w1u_ref, w1i_ref, b1_ref, g1_ref, be1_ref,
              w2_ref, b2_ref, g2_ref, be2_ref,
              w3_ref, b3_ref, g3_ref, be3_ref,
              w4_ref, b4_ref, g4_ref, be4_ref,
              wp_ref, bp_ref, o_ref):
    eu = jnp.where(mu_ref[...] == 0, pu_ref[:, :_D], pu_ref[:, _D:])
    ei = jnp.where(mi_ref[...] == 0, pi_ref[:, :_D], pi_ref[:, _D:])
    x = (jnp.dot(eu, w1u_ref[...], preferred_element_type=jnp.float32)
         + jnp.dot(ei, w1i_ref[...], preferred_element_type=jnp.float32)
         + b1_ref[...])
    x = _ln_relu(x, g1_ref[...], be1_ref[...])
    x = jnp.dot(x, w2_ref[...], preferred_element_type=jnp.float32) + b2_ref[...]
    x = _ln_relu(x, g2_ref[...], be2_ref[...])
    x = jnp.dot(x, w3_ref[...], preferred_element_type=jnp.float32) + b3_ref[...]
    x = _ln_relu(x, g3_ref[...], be3_ref[...])
    x = jnp.dot(x, w4_ref[...], preferred_element_type=jnp.float32) + b4_ref[...]
    x = _ln_relu(x, g4_ref[...], be4_ref[...])
    o_ref[...] = (jnp.dot(x, wp_ref[...], preferred_element_type=jnp.float32)
                  + bp_ref[...])


def _tc_mlp(pu, pi, mu, mi, Ws, bs, gammas, betas, Wp, bp):
    w1u = Ws[0][:_D]
    w1i = Ws[0][_D:]
    row = lambda v: v.reshape(1, -1)

    def full(a):
        return pl.BlockSpec(a.shape, lambda i: (0,) * a.ndim)

    pair_spec = pl.BlockSpec((_TILE, 2 * _D), lambda i: (i, 0))
    mask_spec = pl.BlockSpec((_TILE, 1), lambda i: (i, 0))
    args = (pu, pi, mu, mi,
            w1u, w1i, row(bs[0]), row(gammas[0]), row(betas[0]),
            Ws[1], row(bs[1]), row(gammas[1]), row(betas[1]),
            Ws[2], row(bs[2]), row(gammas[2]), row(betas[2]),
            Ws[3], row(bs[3]), row(gammas[3]), row(betas[3]),
            Wp, row(bp))
    in_specs = [pair_spec, pair_spec, mask_spec, mask_spec]
    in_specs += [full(a) for a in args[4:]]
    out = pl.pallas_call(
        _mlp_body,
        grid=(_B // _TILE,),
        in_specs=in_specs,
        out_specs=pl.BlockSpec((_TILE, 1), lambda i: (i, 0)),
        out_shape=jax.ShapeDtypeStruct((_B, 1), jnp.float32),
    )(*args)
    return out.reshape(_B)


def kernel(users, items, user_table, item_table, Ws, bs, gammas, betas, Wp, bp):
    users = users.astype(jnp.int32)
    items = items.astype(jnp.int32)
    ut2 = user_table.reshape(_V // 2, 2 * _D)
    it2 = item_table.reshape(_V // 2, 2 * _D)
    pu, pi = _sc_gather_pair(ut2, it2,
                             lax.shift_right_logical(users, 1),
                             lax.shift_right_logical(items, 1))
    mu = lax.bitwise_and(users, 1).reshape(_B, 1)
    mi = lax.bitwise_and(items, 1).reshape(_B, 1)
    return _tc_mlp(pu, pi, mu, mi, Ws, bs, gammas, betas, Wp, bp)


# speedup vs baseline: 1.2146x; 1.2146x over previous
import functools

import jax
import jax.numpy as jnp
from jax import lax
from jax.experimental import pallas as pl
from jax.experimental.pallas import tpu as pltpu
from jax.experimental.pallas import tpu_sc as plsc

_B = 16384
_D = 64
_V = 1000000
_NC = 2
_NS = 16
_NW = _NC * _NS
_BPW = _B // _NW

_TILE = 1024
_EPS = 1e-5


def _sc_gather2(comb, users, items):
    mesh = plsc.VectorSubcoreMesh(core_axis_name="c", subcore_axis_name="s")
    out_t = (jax.ShapeDtypeStruct((_B, 2 * _D), jnp.float32),
             jax.ShapeDtypeStruct((_B, 2 * _D), jnp.float32))
    half = _BPW // 2

    @functools.partial(
        pl.kernel, mesh=mesh, out_type=out_t,
        scratch_types=[
            pltpu.VMEM((_BPW,), jnp.int32),
            pltpu.VMEM((_BPW,), jnp.int32),
            pltpu.VMEM((half, 2 * _D), jnp.float32),
            pltpu.VMEM((half, 2 * _D), jnp.float32),
            pltpu.SemaphoreType.DMA,
            pltpu.SemaphoreType.DMA,
        ],
    )
    def k(c_hbm, u_hbm, i_hbm, ou_hbm, oi_hbm,
          uidx_v, iidx_v, rows0_v, rows1_v, sem0, sem1):
        wid = lax.axis_index("s") * _NC + lax.axis_index("c")
        base = wid * _BPW
        pltpu.sync_copy(u_hbm.at[pl.ds(base, _BPW)], uidx_v)
        pltpu.sync_copy(i_hbm.at[pl.ds(base, _BPW)], iidx_v)
        c0 = pltpu.make_async_copy(
            c_hbm.at[uidx_v.at[pl.ds(0, half)]], rows0_v, sem0)
        c1 = pltpu.make_async_copy(
            c_hbm.at[uidx_v.at[pl.ds(half, half)]], rows1_v, sem1)
        c0.start()
        c1.start()
        c0.wait()
        pltpu.sync_copy(rows0_v, ou_hbm.at[pl.ds(base, half)])
        c2 = pltpu.make_async_copy(
            c_hbm.at[iidx_v.at[pl.ds(0, half)]], rows0_v, sem0)
        c2.start()
        c1.wait()
        pltpu.sync_copy(rows1_v, ou_hbm.at[pl.ds(base + half, half)])
        c3 = pltpu.make_async_copy(
            c_hbm.at[iidx_v.at[pl.ds(half, half)]], rows1_v, sem1)
        c3.start()
        c2.wait()
        pltpu.sync_copy(rows0_v, oi_hbm.at[pl.ds(base, half)])
        c3.wait()
        pltpu.sync_copy(rows1_v, oi_hbm.at[pl.ds(base + half, half)])

    return k(comb, users, items)


def _ln_relu(x, g, b):
    mu = jnp.mean(x, axis=-1, keepdims=True)
    xc = x - mu
    var = jnp.mean(xc * xc, axis=-1, keepdims=True)
    y = xc * lax.rsqrt(var + _EPS) * g + b
    return jnp.maximum(y, 0.0)


def _mlp_body(pu_ref, pi_ref,
              w1u_ref, w1i_ref, b1_ref, g1_ref, be1_ref,
              w2_ref, b2_ref, g2_ref, be2_ref,
              w3_ref, b3_ref, g3_ref, be3_ref,
              w4_ref, b4_ref, g4_ref, be4_ref,
              wp_ref, bp_ref, o_ref):
    eu = pu_ref[:, :_D]
    ei = pi_ref[:, _D:]
    x = (jnp.dot(eu, w1u_ref[...], preferred_element_type=jnp.float32)
         + jnp.dot(ei, w1i_ref[...], preferred_element_type=jnp.float32)
         + b1_ref[...])
    x = _ln_relu(x, g1_ref[...], be1_ref[...])
    x = jnp.dot(x, w2_ref[...], preferred_element_type=jnp.float32) + b2_ref[...]
    x = _ln_relu(x, g2_ref[...], be2_ref[...])
    x = jnp.dot(x, w3_ref[...], preferred_element_type=jnp.float32) + b3_ref[...]
    x = _ln_relu(x, g3_ref[...], be3_ref[...])
    x = jnp.dot(x, w4_ref[...], preferred_element_type=jnp.float32) + b4_ref[...]
    x = _ln_relu(x, g4_ref[...], be4_ref[...])
    o_ref[...] = (jnp.dot(x, wp_ref[...], preferred_element_type=jnp.float32)
                  + bp_ref[...])


def _tc_mlp(pu, pi, Ws, bs, gammas, betas, Wp, bp):
    w1u = Ws[0][:_D]
    w1i = Ws[0][_D:]
    row = lambda v: v.reshape(1, -1)

    def full(a):
        return pl.BlockSpec(a.shape, lambda i: (0,) * a.ndim)

    pair_spec = pl.BlockSpec((_TILE, 2 * _D), lambda i: (i, 0))
    args = (pu, pi,
            w1u, w1i, row(bs[0]), row(gammas[0]), row(betas[0]),
            Ws[1], row(bs[1]), row(gammas[1]), row(betas[1]),
            Ws[2], row(bs[2]), row(gammas[2]), row(betas[2]),
            Ws[3], row(bs[3]), row(gammas[3]), row(betas[3]),
            Wp, row(bp))
    in_specs = [pair_spec, pair_spec] + [full(a) for a in args[2:]]
    out = pl.pallas_call(
        _mlp_body,
        grid=(_B // _TILE,),
        in_specs=in_specs,
        out_specs=pl.BlockSpec((_TILE, 1), lambda i: (i, 0)),
        out_shape=jax.ShapeDtypeStruct((_B, 1), jnp.float32),
    )(*args)
    return out.reshape(_B)


def kernel(users, items, user_table, item_table, Ws, bs, gammas, betas, Wp, bp):
    comb = jnp.concatenate([user_table, item_table], axis=1)
    pu, pi = _sc_gather2(comb, users.astype(jnp.int32), items.astype(jnp.int32))
    return _tc_mlp(pu, pi, Ws, bs, gammas, betas, Wp, bp)


# speedup vs baseline: 1.2189x; 1.0035x over previous
import functools

import jax
import jax.numpy as jnp
from jax import lax
from jax.experimental import pallas as pl
from jax.experimental.pallas import tpu as pltpu
from jax.experimental.pallas import tpu_sc as plsc

_B = 16384
_D = 64
_V = 1000000
_NC = 2
_NS = 16
_NW = _NC * _NS
_BPW = _B // _NW

_RC = 1024
_TILE = 1024
_EPS = 1e-5


def _repack_body(utT_ref, itT_ref, o_ref):
    o_ref[:, :_D] = jnp.transpose(utT_ref[...], (1, 0))
    o_ref[:, _D:] = jnp.transpose(itT_ref[...], (1, 0))


def _tc_repack(utT, itT):
    grid = (_V + _RC - 1) // _RC
    return pl.pallas_call(
        _repack_body,
        grid=(grid,),
        in_specs=[pl.BlockSpec((_D, _RC), lambda i: (0, i)),
                  pl.BlockSpec((_D, _RC), lambda i: (0, i))],
        out_specs=pl.BlockSpec((_RC, 2 * _D), lambda i: (i, 0)),
        out_shape=jax.ShapeDtypeStruct((_V, 2 * _D), jnp.float32),
    )(utT, itT)


def _sc_gather2(comb, users, items):
    mesh = plsc.VectorSubcoreMesh(core_axis_name="c", subcore_axis_name="s")
    out_t = (jax.ShapeDtypeStruct((_B, 2 * _D), jnp.float32),
             jax.ShapeDtypeStruct((_B, 2 * _D), jnp.float32))
    half = _BPW // 2

    @functools.partial(
        pl.kernel, mesh=mesh, out_type=out_t,
        scratch_types=[
            pltpu.VMEM((_BPW,), jnp.int32),
            pltpu.VMEM((_BPW,), jnp.int32),
            pltpu.VMEM((half, 2 * _D), jnp.float32),
            pltpu.VMEM((half, 2 * _D), jnp.float32),
            pltpu.SemaphoreType.DMA,
            pltpu.SemaphoreType.DMA,
        ],
    )
    def k(c_hbm, u_hbm, i_hbm, ou_hbm, oi_hbm,
          uidx_v, iidx_v, rows0_v, rows1_v, sem0, sem1):
        wid = lax.axis_index("s") * _NC + lax.axis_index("c")
        base = wid * _BPW
        pltpu.sync_copy(u_hbm.at[pl.ds(base, _BPW)], uidx_v)
        pltpu.sync_copy(i_hbm.at[pl.ds(base, _BPW)], iidx_v)
        c0 = pltpu.make_async_copy(
            c_hbm.at[uidx_v.at[pl.ds(0, half)]], rows0_v, sem0)
        c1 = pltpu.make_async_copy(
            c_hbm.at[uidx_v.at[pl.ds(half, half)]], rows1_v, sem1)
        c0.start()
        c1.start()
        c0.wait()
        pltpu.sync_copy(rows0_v, ou_hbm.at[pl.ds(base, half)])
        c2 = pltpu.make_async_copy(
            c_hbm.at[iidx_v.at[pl.ds(0, half)]], rows0_v, sem0)
        c2.start()
        c1.wait()
        pltpu.sync_copy(rows1_v, ou_hbm.at[pl.ds(base + half, half)])
        c3 = pltpu.make_async_copy(
            c_hbm.at[iidx_v.at[pl.ds(half, half)]], rows1_v, sem1)
        c3.start()
        c2.wait()
        pltpu.sync_copy(rows0_v, oi_hbm.at[pl.ds(base, half)])
        c3.wait()
        pltpu.sync_copy(rows1_v, oi_hbm.at[pl.ds(base + half, half)])

    return k(comb, users, items)


def _ln_relu(x, g, b):
    mu = jnp.mean(x, axis=-1, keepdims=True)
    xc = x - mu
    var = jnp.mean(xc * xc, axis=-1, keepdims=True)
    y = xc * lax.rsqrt(var + _EPS) * g + b
    return jnp.maximum(y, 0.0)


def _mlp_body(pu_ref, pi_ref,
              w1u_ref, w1i_ref, b1_ref, g1_ref, be1_ref,
              w2_ref, b2_ref, g2_ref, be2_ref,
              w3_ref, b3_ref, g3_ref, be3_ref,
              w4_ref, b4_ref, g4_ref, be4_ref,
              wp_ref, bp_ref, o_ref):
    eu = pu_ref[:, :_D]
    ei = pi_ref[:, _D:]
    x = (jnp.dot(eu, w1u_ref[...], preferred_element_type=jnp.float32)
         + jnp.dot(ei, w1i_ref[...], preferred_element_type=jnp.float32)
         + b1_ref[...])
    x = _ln_relu(x, g1_ref[...], be1_ref[...])
    x = jnp.dot(x, w2_ref[...], preferred_element_type=jnp.float32) + b2_ref[...]
    x = _ln_relu(x, g2_ref[...], be2_ref[...])
    x = jnp.dot(x, w3_ref[...], preferred_element_type=jnp.float32) + b3_ref[...]
    x = _ln_relu(x, g3_ref[...], be3_ref[...])
    x = jnp.dot(x, w4_ref[...], preferred_element_type=jnp.float32) + b4_ref[...]
    x = _ln_relu(x, g4_ref[...], be4_ref[...])
    o_ref[...] = (jnp.dot(x, wp_ref[...], preferred_element_type=jnp.float32)
                  + bp_ref[...])


def _tc_mlp(pu, pi, Ws, bs, gammas, betas, Wp, bp):
    w1u = Ws[0][:_D]
    w1i = Ws[0][_D:]
    row = lambda v: v.reshape(1, -1)

    def full(a):
        return pl.BlockSpec(a.shape, lambda i: (0,) * a.ndim)

    pair_spec = pl.BlockSpec((_TILE, 2 * _D), lambda i: (i, 0))
    args = (pu, pi,
            w1u, w1i, row(bs[0]), row(gammas[0]), row(betas[0]),
            Ws[1], row(bs[1]), row(gammas[1]), row(betas[1]),
            Ws[2], row(bs[2]), row(gammas[2]), row(betas[2]),
            Ws[3], row(bs[3]), row(gammas[3]), row(betas[3]),
            Wp, row(bp))
    in_specs = [pair_spec, pair_spec] + [full(a) for a in args[2:]]
    out = pl.pallas_call(
        _mlp_body,
        grid=(_B // _TILE,),
        in_specs=in_specs,
        out_specs=pl.BlockSpec((_TILE, 1), lambda i: (i, 0)),
        out_shape=jax.ShapeDtypeStruct((_B, 1), jnp.float32),
    )(*args)
    return out.reshape(_B)


def kernel(users, items, user_table, item_table, Ws, bs, gammas, betas, Wp, bp):
    utT = jnp.swapaxes(user_table, 0, 1)
    itT = jnp.swapaxes(item_table, 0, 1)
    comb = _tc_repack(utT, itT)
    pu, pi = _sc_gather2(comb, users.astype(jnp.int32), items.astype(jnp.int32))
    return _tc_mlp(pu, pi, Ws, bs, gammas, betas, Wp, bp)


# speedup vs baseline: 2.4949x; 2.0468x over previous
import functools

import jax
import jax.numpy as jnp
from jax import lax
from jax.experimental import pallas as pl
from jax.experimental.pallas import tpu as pltpu
from jax.experimental.pallas import tpu_sc as plsc

_B = 16384
_D = 64
_V = 1000000
_NC = 2
_NS = 16
_NW = _NC * _NS
_BPW = _B // _NW

_RC = 4096
_H = _RC // 2
_NBLK = (_V + _RC - 1) // _RC
_P = _NBLK * _H
_TILE = 1024
_EPS = 1e-5


def _repack_body(utT_ref, itT_ref, eye_ref, o_ref):
    dims = (((0,), (0,)), ((), ()))
    eye = eye_ref[...]

    def pack(xT_ref):
        x16 = xT_ref[...].astype(jnp.bfloat16)
        ylo = lax.dot_general(x16[:, :_H], eye, dims,
                              preferred_element_type=jnp.float32)
        yhi = lax.dot_general(x16[:, _H:], eye, dims,
                              preferred_element_type=jnp.float32)
        lo = lax.shift_right_logical(
            lax.bitcast_convert_type(ylo, jnp.int32), 16)
        hi = lax.bitwise_and(lax.bitcast_convert_type(yhi, jnp.int32),
                             jnp.int32(-65536))
        return lax.bitwise_or(lo, hi)

    o_ref[:, :_D] = pack(utT_ref)
    o_ref[:, _D:] = pack(itT_ref)


def _tc_repack(utT, itT):
    eye = jnp.eye(_D, dtype=jnp.bfloat16)
    return pl.pallas_call(
        _repack_body,
        grid=(_NBLK,),
        in_specs=[pl.BlockSpec((_D, _RC), lambda i: (0, i)),
                  pl.BlockSpec((_D, _RC), lambda i: (0, i)),
                  pl.BlockSpec((_D, _D), lambda i: (0, 0))],
        out_specs=pl.BlockSpec((_H, 2 * _D), lambda i: (i, 0)),
        out_shape=jax.ShapeDtypeStruct((_P, 2 * _D), jnp.int32),
    )(utT, itT, eye)


def _sc_gather2(comb, pu_idx, pi_idx):
    mesh = plsc.VectorSubcoreMesh(core_axis_name="c", subcore_axis_name="s")
    out_t = (jax.ShapeDtypeStruct((_B, 2 * _D), jnp.int32),
             jax.ShapeDtypeStruct((_B, 2 * _D), jnp.int32))
    half = _BPW // 2

    @functools.partial(
        pl.kernel, mesh=mesh, out_type=out_t,
        scratch_types=[
            pltpu.VMEM((_BPW,), jnp.int32),
            pltpu.VMEM((_BPW,), jnp.int32),
            pltpu.VMEM((half, 2 * _D), jnp.int32),
            pltpu.VMEM((half, 2 * _D), jnp.int32),
            pltpu.SemaphoreType.DMA,
            pltpu.SemaphoreType.DMA,
        ],
    )
    def k(c_hbm, u_hbm, i_hbm, ou_hbm, oi_hbm,
          uidx_v, iidx_v, rows0_v, rows1_v, sem0, sem1):
        wid = lax.axis_index("s") * _NC + lax.axis_index("c")
        base = wid * _BPW
        pltpu.sync_copy(u_hbm.at[pl.ds(base, _BPW)], uidx_v)
        pltpu.sync_copy(i_hbm.at[pl.ds(base, _BPW)], iidx_v)
        c0 = pltpu.make_async_copy(
            c_hbm.at[uidx_v.at[pl.ds(0, half)]], rows0_v, sem0)
        c1 = pltpu.make_async_copy(
            c_hbm.at[uidx_v.at[pl.ds(half, half)]], rows1_v, sem1)
        c0.start()
        c1.start()
        c0.wait()
        pltpu.sync_copy(rows0_v, ou_hbm.at[pl.ds(base, half)])
        c2 = pltpu.make_async_copy(
            c_hbm.at[iidx_v.at[pl.ds(0, half)]], rows0_v, sem0)
        c2.start()
        c1.wait()
        pltpu.sync_copy(rows1_v, ou_hbm.at[pl.ds(base + half, half)])
        c3 = pltpu.make_async_copy(
            c_hbm.at[iidx_v.at[pl.ds(half, half)]], rows1_v, sem1)
        c3.start()
        c2.wait()
        pltpu.sync_copy(rows0_v, oi_hbm.at[pl.ds(base, half)])
        c3.wait()
        pltpu.sync_copy(rows1_v, oi_hbm.at[pl.ds(base + half, half)])

    return k(comb, pu_idx, pi_idx)


def _ln_relu(x, g, b):
    mu = jnp.mean(x, axis=-1, keepdims=True)
    xc = x - mu
    var = jnp.mean(xc * xc, axis=-1, keepdims=True)
    y = xc * lax.rsqrt(var + _EPS) * g + b
    return jnp.maximum(y, 0.0)


def _unpack(raw, par):
    sel = jnp.where(par == 1, lax.bitwise_and(raw, jnp.int32(-65536)),
                    lax.shift_left(raw, 16))
    return lax.bitcast_convert_type(sel, jnp.float32)


def _mlp_body(pu_ref, pi_ref, mu_ref, mi_ref,
              w1u_ref, w1i_ref, b1_ref, g1_ref, be1_ref,
              w2_ref, b2_ref, g2_ref, be2_ref,
              w3_ref, b3_ref, g3_ref, be3_ref,
              w4_ref, b4_ref, g4_ref, be4_ref,
              wp_ref, bp_ref, o_ref):
    eu = _unpack(pu_ref[...], mu_ref[...])[:, :_D]
    ei = _unpack(pi_ref[...], mi_ref[...])[:, _D:]
    x = (jnp.dot(eu, w1u_ref[...], preferred_element_type=jnp.float32)
         + jnp.dot(ei, w1i_ref[...], preferred_element_type=jnp.float32)
         + b1_ref[...])
    x = _ln_relu(x, g1_ref[...], be1_ref[...])
    x = jnp.dot(x, w2_ref[...], preferred_element_type=jnp.float32) + b2_ref[...]
    x = _ln_relu(x, g2_ref[...], be2_ref[...])
    x = jnp.dot(x, w3_ref[...], preferred_element_type=jnp.float32) + b3_ref[...]
    x = _ln_relu(x, g3_ref[...], be3_ref[...])
    x = jnp.dot(x, w4_ref[...], preferred_element_type=jnp.float32) + b4_ref[...]
    x = _ln_relu(x, g4_ref[...], be4_ref[...])
    o_ref[...] = (jnp.dot(x, wp_ref[...], preferred_element_type=jnp.float32)
                  + bp_ref[...])


def _tc_mlp(pu, pi, mu, mi, Ws, bs, gammas, betas, Wp, bp):
    w1u = Ws[0][:_D]
    w1i = Ws[0][_D:]
    row = lambda v: v.reshape(1, -1)

    def full(a):
        return pl.BlockSpec(a.shape, lambda i: (0,) * a.ndim)

    pair_spec = pl.BlockSpec((_TILE, 2 * _D), lambda i: (i, 0))
    mask_spec = pl.BlockSpec((_TILE, 1), lambda i: (i, 0))
    args = (pu, pi, mu, mi,
            w1u, w1i, row(bs[0]), row(gammas[0]), row(betas[0]),
            Ws[1], row(bs[1]), row(gammas[1]), row(betas[1]),
            Ws[2], row(bs[2]), row(gammas[2]), row(betas[2]),
            Ws[3], row(bs[3]), row(gammas[3]), row(betas[3]),
            Wp, row(bp))
    in_specs = [pair_spec, pair_spec, mask_spec, mask_spec]
    in_specs += [full(a) for a in args[4:]]
    out = pl.pallas_call(
        _mlp_body,
        grid=(_B // _TILE,),
        in_specs=in_specs,
        out_specs=pl.BlockSpec((_TILE, 1), lambda i: (i, 0)),
        out_shape=jax.ShapeDtypeStruct((_B, 1), jnp.float32),
    )(*args)
    return out.reshape(_B)


def _packed_idx(v):
    rem = lax.bitwise_and(v, _RC - 1)
    blk = lax.shift_right_logical(v, 12)
    p = lax.bitwise_or(lax.shift_left(blk, 11),
                       lax.bitwise_and(rem, _H - 1))
    par = lax.shift_right_logical(rem, 11)
    return p, par.reshape(_B, 1)


def kernel(users, items, user_table, item_table, Ws, bs, gammas, betas, Wp, bp):
    users = users.astype(jnp.int32)
    items = items.astype(jnp.int32)
    utT = jnp.swapaxes(user_table, 0, 1)
    itT = jnp.swapaxes(item_table, 0, 1)
    comb = _tc_repack(utT, itT)
    pu_idx, mu = _packed_idx(users)
    pi_idx, mi = _packed_idx(items)
    pu, pi = _sc_gather2(comb, pu_idx, pi_idx)
    return _tc_mlp(pu, pi, mu, mi, Ws, bs, gammas, betas, Wp, bp)


# speedup vs baseline: 2.9942x; 1.2001x over previous
import functools

import jax
import jax.numpy as jnp
from jax import lax
from jax.experimental import pallas as pl
from jax.experimental.pallas import tpu as pltpu
from jax.experimental.pallas import tpu_sc as plsc

_B = 16384
_D = 64
_V = 1000000
_NC = 2
_NS = 16
_NW = _NC * _NS
_BPW = _B // _NW

_RC = 8192
_H = _RC // 2
_NBLK = (_V + _RC - 1) // _RC
_P = _NBLK * _H
_TILE = 1024
_EPS = 1e-5


def _repack_body(utT_ref, itT_ref, eye_ref, o_ref):
    dims = (((0,), (0,)), ((), ()))
    eye = eye_ref[...]

    def pack(xT_ref):
        x16 = xT_ref[...].astype(jnp.bfloat16)
        ylo = lax.dot_general(x16[:, :_H], eye, dims,
                              preferred_element_type=jnp.float32)
        yhi = lax.dot_general(x16[:, _H:], eye, dims,
                              preferred_element_type=jnp.float32)
        lo = lax.shift_right_logical(
            lax.bitcast_convert_type(ylo, jnp.int32), 16)
        hi = lax.bitwise_and(lax.bitcast_convert_type(yhi, jnp.int32),
                             jnp.int32(-65536))
        return lax.bitwise_or(lo, hi)

    o_ref[:, :_D] = pack(utT_ref)
    o_ref[:, _D:] = pack(itT_ref)


def _tc_repack(utT, itT):
    eye = jnp.eye(_D, dtype=jnp.bfloat16)
    return pl.pallas_call(
        _repack_body,
        grid=(_NBLK,),
        in_specs=[pl.BlockSpec((_D, _RC), lambda i: (0, i)),
                  pl.BlockSpec((_D, _RC), lambda i: (0, i)),
                  pl.BlockSpec((_D, _D), lambda i: (0, 0))],
        out_specs=pl.BlockSpec((_H, 2 * _D), lambda i: (i, 0)),
        out_shape=jax.ShapeDtypeStruct((_P, 2 * _D), jnp.int32),
    )(utT, itT, eye)


def _sc_gather2(comb, pu_idx, pi_idx):
    mesh = plsc.VectorSubcoreMesh(core_axis_name="c", subcore_axis_name="s")
    out_t = (jax.ShapeDtypeStruct((_B, 2 * _D), jnp.int32),
             jax.ShapeDtypeStruct((_B, 2 * _D), jnp.int32))
    half = _BPW // 2

    @functools.partial(
        pl.kernel, mesh=mesh, out_type=out_t,
        scratch_types=[
            pltpu.VMEM((_BPW,), jnp.int32),
            pltpu.VMEM((_BPW,), jnp.int32),
            pltpu.VMEM((half, 2 * _D), jnp.int32),
            pltpu.VMEM((half, 2 * _D), jnp.int32),
            pltpu.SemaphoreType.DMA,
            pltpu.SemaphoreType.DMA,
        ],
    )
    def k(c_hbm, u_hbm, i_hbm, ou_hbm, oi_hbm,
          uidx_v, iidx_v, rows0_v, rows1_v, sem0, sem1):
        wid = lax.axis_index("s") * _NC + lax.axis_index("c")
        base = wid * _BPW
        pltpu.sync_copy(u_hbm.at[pl.ds(base, _BPW)], uidx_v)
        pltpu.sync_copy(i_hbm.at[pl.ds(base, _BPW)], iidx_v)
        c0 = pltpu.make_async_copy(
            c_hbm.at[uidx_v.at[pl.ds(0, half)]], rows0_v, sem0)
        c1 = pltpu.make_async_copy(
            c_hbm.at[uidx_v.at[pl.ds(half, half)]], rows1_v, sem1)
        c0.start()
        c1.start()
        c0.wait()
        pltpu.sync_copy(rows0_v, ou_hbm.at[pl.ds(base, half)])
        c2 = pltpu.make_async_copy(
            c_hbm.at[iidx_v.at[pl.ds(0, half)]], rows0_v, sem0)
        c2.start()
        c1.wait()
        pltpu.sync_copy(rows1_v, ou_hbm.at[pl.ds(base + half, half)])
        c3 = pltpu.make_async_copy(
            c_hbm.at[iidx_v.at[pl.ds(half, half)]], rows1_v, sem1)
        c3.start()
        c2.wait()
        pltpu.sync_copy(rows0_v, oi_hbm.at[pl.ds(base, half)])
        c3.wait()
        pltpu.sync_copy(rows1_v, oi_hbm.at[pl.ds(base + half, half)])

    return k(comb, pu_idx, pi_idx)


def _ln_relu(x, g, b):
    mu = jnp.mean(x, axis=-1, keepdims=True)
    xc = x - mu
    var = jnp.mean(xc * xc, axis=-1, keepdims=True)
    y = xc * lax.rsqrt(var + _EPS) * g + b
    return jnp.maximum(y, 0.0)


def _unpack(raw, par):
    sel = jnp.where(par == 1, lax.bitwise_and(raw, jnp.int32(-65536)),
                    lax.shift_left(raw, 16))
    return lax.bitcast_convert_type(sel, jnp.float32)


def _mlp_body(pu_ref, pi_ref, mu_ref, mi_ref,
              w1u_ref, w1i_ref, b1_ref, g1_ref, be1_ref,
              w2_ref, b2_ref, g2_ref, be2_ref,
              w3_ref, b3_ref, g3_ref, be3_ref,
              w4_ref, b4_ref, g4_ref, be4_ref,
              wp_ref, bp_ref, o_ref):
    eu = _unpack(pu_ref[...], mu_ref[...])[:, :_D]
    ei = _unpack(pi_ref[...], mi_ref[...])[:, _D:]
    x = (jnp.dot(eu, w1u_ref[...], preferred_element_type=jnp.float32)
         + jnp.dot(ei, w1i_ref[...], preferred_element_type=jnp.float32)
         + b1_ref[...])
    x = _ln_relu(x, g1_ref[...], be1_ref[...])
    x = jnp.dot(x, w2_ref[...], preferred_element_type=jnp.float32) + b2_ref[...]
    x = _ln_relu(x, g2_ref[...], be2_ref[...])
    x = jnp.dot(x, w3_ref[...], preferred_element_type=jnp.float32) + b3_ref[...]
    x = _ln_relu(x, g3_ref[...], be3_ref[...])
    x = jnp.dot(x, w4_ref[...], preferred_element_type=jnp.float32) + b4_ref[...]
    x = _ln_relu(x, g4_ref[...], be4_ref[...])
    o_ref[...] = (jnp.dot(x, wp_ref[...], preferred_element_type=jnp.float32)
                  + bp_ref[...])


def _tc_mlp(pu, pi, mu, mi, Ws, bs, gammas, betas, Wp, bp):
    w1u = Ws[0][:_D]
    w1i = Ws[0][_D:]
    row = lambda v: v.reshape(1, -1)

    def full(a):
        return pl.BlockSpec(a.shape, lambda i: (0,) * a.ndim)

    pair_spec = pl.BlockSpec((_TILE, 2 * _D), lambda i: (i, 0))
    mask_spec = pl.BlockSpec((_TILE, 1), lambda i: (i, 0))
    args = (pu, pi, mu, mi,
            w1u, w1i, row(bs[0]), row(gammas[0]), row(betas[0]),
            Ws[1], row(bs[1]), row(gammas[1]), row(betas[1]),
            Ws[2], row(bs[2]), row(gammas[2]), row(betas[2]),
            Ws[3], row(bs[3]), row(gammas[3]), row(betas[3]),
            Wp, row(bp))
    in_specs = [pair_spec, pair_spec, mask_spec, mask_spec]
    in_specs += [full(a) for a in args[4:]]
    out = pl.pallas_call(
        _mlp_body,
        grid=(_B // _TILE,),
        in_specs=in_specs,
        out_specs=pl.BlockSpec((_TILE, 1), lambda i: (i, 0)),
        out_shape=jax.ShapeDtypeStruct((_B, 1), jnp.float32),
    )(*args)
    return out.reshape(_B)


def _packed_idx(v):
    rem = lax.bitwise_and(v, _RC - 1)
    blk = lax.shift_right_logical(v, 13)
    p = lax.bitwise_or(lax.shift_left(blk, 12),
                       lax.bitwise_and(rem, _H - 1))
    par = lax.shift_right_logical(rem, 12)
    return p, par.reshape(_B, 1)


def kernel(users, items, user_table, item_table, Ws, bs, gammas, betas, Wp, bp):
    users = users.astype(jnp.int32)
    items = items.astype(jnp.int32)
    utT = jnp.swapaxes(user_table, 0, 1)
    itT = jnp.swapaxes(item_table, 0, 1)
    comb = _tc_repack(utT, itT)
    pu_idx, mu = _packed_idx(users)
    pi_idx, mi = _packed_idx(items)
    pu, pi = _sc_gather2(comb, pu_idx, pi_idx)
    return _tc_mlp(pu, pi, mu, mi, Ws, bs, gammas, betas, Wp, bp)


# speedup vs baseline: 3.2835x; 1.0966x over previous
import functools

import jax
import jax.numpy as jnp
from jax import lax
from jax.experimental import pallas as pl
from jax.experimental.pallas import tpu as pltpu
from jax.experimental.pallas import tpu_sc as plsc

_B = 16384
_D = 64
_V = 1000000
_NC = 2
_NS = 16
_NW = _NC * _NS
_BPW = _B // _NW

_RC = 16384
_H = _RC // 2
_NBLK = (_V + _RC - 1) // _RC
_P = _NBLK * _H
_TILE = 1024
_EPS = 1e-5


def _repack_body(utT_ref, itT_ref, eye_ref, o_ref):
    dims = (((0,), (0,)), ((), ()))
    eye = eye_ref[...]

    def pack(xT_ref):
        x16 = xT_ref[...].astype(jnp.bfloat16)
        ylo = lax.dot_general(x16[:, :_H], eye, dims,
                              preferred_element_type=jnp.float32)
        yhi = lax.dot_general(x16[:, _H:], eye, dims,
                              preferred_element_type=jnp.float32)
        lo = lax.shift_right_logical(
            lax.bitcast_convert_type(ylo, jnp.int32), 16)
        hi = lax.bitwise_and(lax.bitcast_convert_type(yhi, jnp.int32),
                             jnp.int32(-65536))
        return lax.bitwise_or(lo, hi)

    o_ref[:, :_D] = pack(utT_ref)
    o_ref[:, _D:] = pack(itT_ref)


def _tc_repack(utT, itT):
    eye = jnp.eye(_D, dtype=jnp.bfloat16)
    return pl.pallas_call(
        _repack_body,
        grid=(_NBLK,),
        in_specs=[pl.BlockSpec((_D, _RC), lambda i: (0, i)),
                  pl.BlockSpec((_D, _RC), lambda i: (0, i)),
                  pl.BlockSpec((_D, _D), lambda i: (0, 0))],
        out_specs=pl.BlockSpec((_H, 2 * _D), lambda i: (i, 0)),
        out_shape=jax.ShapeDtypeStruct((_P, 2 * _D), jnp.int32),
    )(utT, itT, eye)


def _sc_gather2(comb, pu_idx, pi_idx):
    mesh = plsc.VectorSubcoreMesh(core_axis_name="c", subcore_axis_name="s")
    out_t = (jax.ShapeDtypeStruct((_B, 2 * _D), jnp.int32),
             jax.ShapeDtypeStruct((_B, 2 * _D), jnp.int32))
    half = _BPW // 2

    @functools.partial(
        pl.kernel, mesh=mesh, out_type=out_t,
        scratch_types=[
            pltpu.VMEM((_BPW,), jnp.int32),
            pltpu.VMEM((_BPW,), jnp.int32),
            pltpu.VMEM((half, 2 * _D), jnp.int32),
            pltpu.VMEM((half, 2 * _D), jnp.int32),
            pltpu.SemaphoreType.DMA,
            pltpu.SemaphoreType.DMA,
        ],
    )
    def k(c_hbm, u_hbm, i_hbm, ou_hbm, oi_hbm,
          uidx_v, iidx_v, rows0_v, rows1_v, sem0, sem1):
        wid = lax.axis_index("s") * _NC + lax.axis_index("c")
        base = wid * _BPW
        pltpu.sync_copy(u_hbm.at[pl.ds(base, _BPW)], uidx_v)
        pltpu.sync_copy(i_hbm.at[pl.ds(base, _BPW)], iidx_v)
        c0 = pltpu.make_async_copy(
            c_hbm.at[uidx_v.at[pl.ds(0, half)]], rows0_v, sem0)
        c1 = pltpu.make_async_copy(
            c_hbm.at[uidx_v.at[pl.ds(half, half)]], rows1_v, sem1)
        c0.start()
        c1.start()
        c0.wait()
        pltpu.sync_copy(rows0_v, ou_hbm.at[pl.ds(base, half)])
        c2 = pltpu.make_async_copy(
            c_hbm.at[iidx_v.at[pl.ds(0, half)]], rows0_v, sem0)
        c2.start()
        c1.wait()
        pltpu.sync_copy(rows1_v, ou_hbm.at[pl.ds(base + half, half)])
        c3 = pltpu.make_async_copy(
            c_hbm.at[iidx_v.at[pl.ds(half, half)]], rows1_v, sem1)
        c3.start()
        c2.wait()
        pltpu.sync_copy(rows0_v, oi_hbm.at[pl.ds(base, half)])
        c3.wait()
        pltpu.sync_copy(rows1_v, oi_hbm.at[pl.ds(base + half, half)])

    return k(comb, pu_idx, pi_idx)


def _ln_relu(x, g, b):
    mu = jnp.mean(x, axis=-1, keepdims=True)
    xc = x - mu
    var = jnp.mean(xc * xc, axis=-1, keepdims=True)
    y = xc * lax.rsqrt(var + _EPS) * g + b
    return jnp.maximum(y, 0.0)


def _unpack(raw, par):
    sel = jnp.where(par == 1, lax.bitwise_and(raw, jnp.int32(-65536)),
                    lax.shift_left(raw, 16))
    return lax.bitcast_convert_type(sel, jnp.float32)


def _mlp_body(pu_ref, pi_ref, mu_ref, mi_ref,
              w1u_ref, w1i_ref, b1_ref, g1_ref, be1_ref,
              w2_ref, b2_ref, g2_ref, be2_ref,
              w3_ref, b3_ref, g3_ref, be3_ref,
              w4_ref, b4_ref, g4_ref, be4_ref,
              wp_ref, bp_ref, o_ref):
    eu = _unpack(pu_ref[...], mu_ref[...])[:, :_D]
    ei = _unpack(pi_ref[...], mi_ref[...])[:, _D:]
    x = (jnp.dot(eu, w1u_ref[...], preferred_element_type=jnp.float32)
         + jnp.dot(ei, w1i_ref[...], preferred_element_type=jnp.float32)
         + b1_ref[...])
    x = _ln_relu(x, g1_ref[...], be1_ref[...])
    x = jnp.dot(x, w2_ref[...], preferred_element_type=jnp.float32) + b2_ref[...]
    x = _ln_relu(x, g2_ref[...], be2_ref[...])
    x = jnp.dot(x, w3_ref[...], preferred_element_type=jnp.float32) + b3_ref[...]
    x = _ln_relu(x, g3_ref[...], be3_ref[...])
    x = jnp.dot(x, w4_ref[...], preferred_element_type=jnp.float32) + b4_ref[...]
    x = _ln_relu(x, g4_ref[...], be4_ref[...])
    o_ref[...] = (jnp.dot(x, wp_ref[...], preferred_element_type=jnp.float32)
                  + bp_ref[...])


def _tc_mlp(pu, pi, mu, mi, Ws, bs, gammas, betas, Wp, bp):
    w1u = Ws[0][:_D]
    w1i = Ws[0][_D:]
    row = lambda v: v.reshape(1, -1)

    def full(a):
        return pl.BlockSpec(a.shape, lambda i: (0,) * a.ndim)

    pair_spec = pl.BlockSpec((_TILE, 2 * _D), lambda i: (i, 0))
    mask_spec = pl.BlockSpec((_TILE, 1), lambda i: (i, 0))
    args = (pu, pi, mu, mi,
            w1u, w1i, row(bs[0]), row(gammas[0]), row(betas[0]),
            Ws[1], row(bs[1]), row(gammas[1]), row(betas[1]),
            Ws[2], row(bs[2]), row(gammas[2]), row(betas[2]),
            Ws[3], row(bs[3]), row(gammas[3]), row(betas[3]),
            Wp, row(bp))
    in_specs = [pair_spec, pair_spec, mask_spec, mask_spec]
    in_specs += [full(a) for a in args[4:]]
    out = pl.pallas_call(
        _mlp_body,
        grid=(_B // _TILE,),
        in_specs=in_specs,
        out_specs=pl.BlockSpec((_TILE, 1), lambda i: (i, 0)),
        out_shape=jax.ShapeDtypeStruct((_B, 1), jnp.float32),
    )(*args)
    return out.reshape(_B)


def _packed_idx(v):
    rem = lax.bitwise_and(v, _RC - 1)
    blk = lax.shift_right_logical(v, 14)
    p = lax.bitwise_or(lax.shift_left(blk, 13),
                       lax.bitwise_and(rem, _H - 1))
    par = lax.shift_right_logical(rem, 13)
    return p, par.reshape(_B, 1)


def kernel(users, items, user_table, item_table, Ws, bs, gammas, betas, Wp, bp):
    users = users.astype(jnp.int32)
    items = items.astype(jnp.int32)
    utT = jnp.swapaxes(user_table, 0, 1)
    itT = jnp.swapaxes(item_table, 0, 1)
    comb = _tc_repack(utT, itT)
    pu_idx, mu = _packed_idx(users)
    pi_idx, mi = _packed_idx(items)
    pu, pi = _sc_gather2(comb, pu_idx, pi_idx)
    return _tc_mlp(pu, pi, mu, mi, Ws, bs, gammas, betas, Wp, bp)


# speedup vs baseline: 3.4333x; 1.0456x over previous
import functools

import jax
import jax.numpy as jnp
from jax import lax
from jax.experimental import pallas as pl
from jax.experimental.pallas import tpu as pltpu
from jax.experimental.pallas import tpu_sc as plsc

_B = 16384
_D = 64
_V = 1000000
_NC = 2
_NS = 16
_NW = _NC * _NS
_BPW = _B // _NW

_RC = 32768
_H = _RC // 2
_NBLK = (_V + _RC - 1) // _RC
_P = _NBLK * _H
_TILE = 1024
_EPS = 1e-5


def _repack_body(utT_ref, itT_ref, eye_ref, o_ref):
    dims = (((0,), (0,)), ((), ()))
    eye = eye_ref[...]

    def pack(xT_ref):
        x16 = xT_ref[...].astype(jnp.bfloat16)
        ylo = lax.dot_general(x16[:, :_H], eye, dims,
                              preferred_element_type=jnp.float32)
        yhi = lax.dot_general(x16[:, _H:], eye, dims,
                              preferred_element_type=jnp.float32)
        lo = lax.shift_right_logical(
            lax.bitcast_convert_type(ylo, jnp.int32), 16)
        hi = lax.bitwise_and(lax.bitcast_convert_type(yhi, jnp.int32),
                             jnp.int32(-65536))
        return lax.bitwise_or(lo, hi)

    o_ref[:, :_D] = pack(utT_ref)
    o_ref[:, _D:] = pack(itT_ref)


def _tc_repack(utT, itT):
    eye = jnp.eye(_D, dtype=jnp.bfloat16)
    return pl.pallas_call(
        _repack_body,
        grid=(_NBLK,),
        in_specs=[pl.BlockSpec((_D, _RC), lambda i: (0, i)),
                  pl.BlockSpec((_D, _RC), lambda i: (0, i)),
                  pl.BlockSpec((_D, _D), lambda i: (0, 0))],
        out_specs=pl.BlockSpec((_H, 2 * _D), lambda i: (i, 0)),
        out_shape=jax.ShapeDtypeStruct((_P, 2 * _D), jnp.int32),
    )(utT, itT, eye)


def _sc_gather2(comb, pu_idx, pi_idx):
    mesh = plsc.VectorSubcoreMesh(core_axis_name="c", subcore_axis_name="s")
    out_t = (jax.ShapeDtypeStruct((_B, 2 * _D), jnp.int32),
             jax.ShapeDtypeStruct((_B, 2 * _D), jnp.int32))
    half = _BPW // 2

    @functools.partial(
        pl.kernel, mesh=mesh, out_type=out_t,
        scratch_types=[
            pltpu.VMEM((_BPW,), jnp.int32),
            pltpu.VMEM((_BPW,), jnp.int32),
            pltpu.VMEM((half, 2 * _D), jnp.int32),
            pltpu.VMEM((half, 2 * _D), jnp.int32),
            pltpu.SemaphoreType.DMA,
            pltpu.SemaphoreType.DMA,
        ],
    )
    def k(c_hbm, u_hbm, i_hbm, ou_hbm, oi_hbm,
          uidx_v, iidx_v, rows0_v, rows1_v, sem0, sem1):
        wid = lax.axis_index("s") * _NC + lax.axis_index("c")
        base = wid * _BPW
        pltpu.sync_copy(u_hbm.at[pl.ds(base, _BPW)], uidx_v)
        pltpu.sync_copy(i_hbm.at[pl.ds(base, _BPW)], iidx_v)
        c0 = pltpu.make_async_copy(
            c_hbm.at[uidx_v.at[pl.ds(0, half)]], rows0_v, sem0)
        c1 = pltpu.make_async_copy(
            c_hbm.at[uidx_v.at[pl.ds(half, half)]], rows1_v, sem1)
        c0.start()
        c1.start()
        c0.wait()
        pltpu.sync_copy(rows0_v, ou_hbm.at[pl.ds(base, half)])
        c2 = pltpu.make_async_copy(
            c_hbm.at[iidx_v.at[pl.ds(0, half)]], rows0_v, sem0)
        c2.start()
        c1.wait()
        pltpu.sync_copy(rows1_v, ou_hbm.at[pl.ds(base + half, half)])
        c3 = pltpu.make_async_copy(
            c_hbm.at[iidx_v.at[pl.ds(half, half)]], rows1_v, sem1)
        c3.start()
        c2.wait()
        pltpu.sync_copy(rows0_v, oi_hbm.at[pl.ds(base, half)])
        c3.wait()
        pltpu.sync_copy(rows1_v, oi_hbm.at[pl.ds(base + half, half)])

    return k(comb, pu_idx, pi_idx)


def _ln_relu(x, g, b):
    mu = jnp.mean(x, axis=-1, keepdims=True)
    xc = x - mu
    var = jnp.mean(xc * xc, axis=-1, keepdims=True)
    y = xc * lax.rsqrt(var + _EPS) * g + b
    return jnp.maximum(y, 0.0)


def _unpack(raw, par):
    sel = jnp.where(par == 1, lax.bitwise_and(raw, jnp.int32(-65536)),
                    lax.shift_left(raw, 16))
    return lax.bitcast_convert_type(sel, jnp.float32)


def _mlp_body(pu_ref, pi_ref, mu_ref, mi_ref,
              w1u_ref, w1i_ref, b1_ref, g1_ref, be1_ref,
              w2_ref, b2_ref, g2_ref, be2_ref,
              w3_ref, b3_ref, g3_ref, be3_ref,
              w4_ref, b4_ref, g4_ref, be4_ref,
              wp_ref, bp_ref, o_ref):
    eu = _unpack(pu_ref[...], mu_ref[...])[:, :_D]
    ei = _unpack(pi_ref[...], mi_ref[...])[:, _D:]
    x = (jnp.dot(eu, w1u_ref[...], preferred_element_type=jnp.float32)
         + jnp.dot(ei, w1i_ref[...], preferred_element_type=jnp.float32)
         + b1_ref[...])
    x = _ln_relu(x, g1_ref[...], be1_ref[...])
    x = jnp.dot(x, w2_ref[...], preferred_element_type=jnp.float32) + b2_ref[...]
    x = _ln_relu(x, g2_ref[...], be2_ref[...])
    x = jnp.dot(x, w3_ref[...], preferred_element_type=jnp.float32) + b3_ref[...]
    x = _ln_relu(x, g3_ref[...], be3_ref[...])
    x = jnp.dot(x, w4_ref[...], preferred_element_type=jnp.float32) + b4_ref[...]
    x = _ln_relu(x, g4_ref[...], be4_ref[...])
    o_ref[...] = (jnp.dot(x, wp_ref[...], preferred_element_type=jnp.float32)
                  + bp_ref[...])


def _tc_mlp(pu, pi, mu, mi, Ws, bs, gammas, betas, Wp, bp):
    w1u = Ws[0][:_D]
    w1i = Ws[0][_D:]
    row = lambda v: v.reshape(1, -1)

    def full(a):
        return pl.BlockSpec(a.shape, lambda i: (0,) * a.ndim)

    pair_spec = pl.BlockSpec((_TILE, 2 * _D), lambda i: (i, 0))
    mask_spec = pl.BlockSpec((_TILE, 1), lambda i: (i, 0))
    args = (pu, pi, mu, mi,
            w1u, w1i, row(bs[0]), row(gammas[0]), row(betas[0]),
            Ws[1], row(bs[1]), row(gammas[1]), row(betas[1]),
            Ws[2], row(bs[2]), row(gammas[2]), row(betas[2]),
            Ws[3], row(bs[3]), row(gammas[3]), row(betas[3]),
            Wp, row(bp))
    in_specs = [pair_spec, pair_spec, mask_spec, mask_spec]
    in_specs += [full(a) for a in args[4:]]
    out = pl.pallas_call(
        _mlp_body,
        grid=(_B // _TILE,),
        in_specs=in_specs,
        out_specs=pl.BlockSpec((_TILE, 1), lambda i: (i, 0)),
        out_shape=jax.ShapeDtypeStruct((_B, 1), jnp.float32),
    )(*args)
    return out.reshape(_B)


def _packed_idx(v):
    rem = lax.bitwise_and(v, _RC - 1)
    blk = lax.shift_right_logical(v, 15)
    p = lax.bitwise_or(lax.shift_left(blk, 14),
                       lax.bitwise_and(rem, _H - 1))
    par = lax.shift_right_logical(rem, 14)
    return p, par.reshape(_B, 1)


def kernel(users, items, user_table, item_table, Ws, bs, gammas, betas, Wp, bp):
    users = users.astype(jnp.int32)
    items = items.astype(jnp.int32)
    utT = jnp.swapaxes(user_table, 0, 1)
    itT = jnp.swapaxes(item_table, 0, 1)
    comb = _tc_repack(utT, itT)
    pu_idx, mu = _packed_idx(users)
    pi_idx, mi = _packed_idx(items)
    pu, pi = _sc_gather2(comb, pu_idx, pi_idx)
    return _tc_mlp(pu, pi, mu, mi, Ws, bs, gammas, betas, Wp, bp)
